# Initial kernel scaffold; baseline (speedup 1.0000x reference)
import jax
import jax.numpy as jnp
from jax.experimental import pallas as pl


def kernel(x, edge_index, batch, W1, b1, W2, b2, W3, b3, lw1, lb1, gamma, beta, lw2, lb2):
    raise NotImplementedError("write your pallas kernel here")


# speedup vs baseline: 8.1861x; 8.1861x over previous
import functools

import jax
import jax.numpy as jnp
import numpy as np
from jax import lax
from jax.experimental import pallas as pl
from jax.experimental.pallas import tpu as pltpu
from jax.experimental.pallas import tpu_sc as plsc

NN = 10000
EE = 320000
NSEG = 64
NC = 2
NS = 16
NW = NC * NS
CHUNK = 128
EPAD = 327680
NCH_E = EPAD // NW // CHUNK
NCH_F = EPAD // NS // CHUNK
SUP = 8
NSUP_E = NCH_E // SUP
NSUP_F = NCH_F // SUP
NPAD = 10240
ZR = NPAD // NS
ZCP = ZR // CHUNK
NB = 1000
GRID = NN // NB
FW = 128


def _make_deg():
    mesh = plsc.VectorSubcoreMesh(core_axis_name="c", subcore_axis_name="s")

    @functools.partial(
        pl.kernel,
        out_type=jax.ShapeDtypeStruct((NC, NPAD, FW), jnp.float32),
        mesh=mesh,
        scratch_types=[
            pltpu.VMEM((NCH_E, CHUNK), jnp.int32),
            pltpu.VMEM((CHUNK, FW), jnp.float32),
            pltpu.VMEM_SHARED((NPAD, FW), jnp.float32),
        ],
    )
    def degk(dstw, ones_h, zeros_h, out, didx, ones_v, acc):
        c = lax.axis_index("c")
        s = lax.axis_index("s")
        w = c * NS + s
        pltpu.sync_copy(dstw.at[w], didx)
        pltpu.sync_copy(ones_h, ones_v)
        for z in range(ZCP):
            pltpu.sync_copy(zeros_h, acc.at[pl.ds(s * ZR + z * CHUNK, CHUNK)])
        plsc.subcore_barrier()

        def body(j, carry):
            pltpu.sync_copy(ones_v, acc.at[didx.at[j]], add=True)
            return carry

        lax.fori_loop(0, NCH_E, body, 0)
        plsc.subcore_barrier()
        pltpu.sync_copy(acc.at[pl.ds(s * ZR, ZR)], out.at[c, pl.ds(s * ZR, ZR)])

    return degk


def _make_scatter(nsup, feature_split):
    mesh = plsc.VectorSubcoreMesh(core_axis_name="c", subcore_axis_name="s")

    @functools.partial(
        pl.kernel,
        out_type=jax.ShapeDtypeStruct((NC, NPAD, FW), jnp.float32),
        mesh=mesh,
        scratch_types=[
            pltpu.VMEM((2, SUP, CHUNK), jnp.int32),
            pltpu.VMEM((2, SUP, CHUNK), jnp.int32),
            pltpu.VMEM((CHUNK, FW), jnp.float32),
            pltpu.VMEM((CHUNK, FW), jnp.float32),
            pltpu.VMEM_SHARED((NPAD, FW), jnp.float32),
            pltpu.SemaphoreType.DMA,
            pltpu.SemaphoreType.DMA,
        ],
    )
    def sck(gtab0, gtab1, srcw, dstw, zeros_h, out,
            sidxb, didxb, rowa, rowb, acc, sa, sb):
        c = lax.axis_index("c")
        s = lax.axis_index("s")
        w = s if feature_split else c * NS + s

        def gather(p, k, row, sem):
            idx = sidxb.at[p, k]
            if feature_split:
                @pl.when(c == 0)
                def _g0():
                    pltpu.async_copy(gtab0.at[idx], row, sem)

                @pl.when(c == 1)
                def _g1():
                    pltpu.async_copy(gtab1.at[idx], row, sem)
            else:
                pltpu.async_copy(gtab0.at[idx], row, sem)

        pltpu.sync_copy(srcw.at[w, 0], sidxb.at[0])
        pltpu.sync_copy(dstw.at[w, 0], didxb.at[0])
        for z in range(ZCP):
            pltpu.sync_copy(zeros_h, acc.at[pl.ds(s * ZR + z * CHUNK, CHUNK)])
        plsc.subcore_barrier()

        gather(0, 0, rowa, sa)
        gather(0, 1, rowb, sb)
        pltpu.sync_copy(srcw.at[w, 1], sidxb.at[1])
        pltpu.sync_copy(dstw.at[w, 1], didxb.at[1])

        def outer(sc, carry):
            p = lax.rem(sc, 2)
            pn = 1 - p
            for k in range(SUP):
                row, sem = (rowa, sa) if k % 2 == 0 else (rowb, sb)
                pltpu.make_async_copy(gtab0.at[sidxb.at[p, k]], row, sem).wait()
                pltpu.sync_copy(row, acc.at[didxb.at[p, k]], add=True)
                if k < SUP - 2:
                    gather(p, k + 2, row, sem)
                else:
                    @pl.when(sc < nsup - 1)
                    def _pf(row=row, sem=sem, k=k):
                        gather(pn, k + 2 - SUP, row, sem)

            @pl.when(sc < nsup - 2)
            def _pidx():
                pltpu.sync_copy(srcw.at[w, sc + 2], sidxb.at[p])
                pltpu.sync_copy(dstw.at[w, sc + 2], didxb.at[p])

            return carry

        lax.fori_loop(0, nsup, outer, 0)
        plsc.subcore_barrier()
        pltpu.sync_copy(acc.at[pl.ds(s * ZR, ZR)], out.at[c, pl.ds(s * ZR, ZR)])

    return sck


_DEG = _make_deg()
_SCAT_E = _make_scatter(NSUP_E, False)
_SCAT_F = _make_scatter(NSUP_F, True)


def _pre_body(x_ref, w1_ref, deg_ref, g_ref, dinv_ref):
    deg = (deg_ref[0] + deg_ref[1])[:, :1] + 1.0
    dinv = lax.rsqrt(deg)
    h = jnp.dot(x_ref[...], w1_ref[...], preferred_element_type=jnp.float32)
    g_ref[...] = jnp.concatenate(
        [h * dinv, jnp.zeros((NB, 64), jnp.float32)], axis=1)
    dinv_ref[...] = dinv


def _pre_call(x, W1, degraw):
    return pl.pallas_call(
        _pre_body,
        grid=(GRID,),
        in_specs=[
            pl.BlockSpec((NB, 128), lambda i: (i, 0)),
            pl.BlockSpec((128, 64), lambda i: (0, 0)),
            pl.BlockSpec((NC, NB, FW), lambda i: (0, i, 0)),
        ],
        out_specs=[
            pl.BlockSpec((NB, FW), lambda i: (i, 0)),
            pl.BlockSpec((NB, 1), lambda i: (i, 0)),
        ],
        out_shape=[
            jax.ShapeDtypeStruct((NN, FW), jnp.float32),
            jax.ShapeDtypeStruct((NN, 1), jnp.float32),
        ],
    )(x, W1, degraw)


def _mid1_body(acc_ref, g_ref, dinv_ref, b_ref, w_ref, out_ref):
    dinv = dinv_ref[...]
    a = (acc_ref[0] + acc_ref[1] + g_ref[...])[:, :64]
    a = jnp.maximum(dinv * a + b_ref[...], 0.0)
    h = jnp.dot(a, w_ref[...], preferred_element_type=jnp.float32)
    out_ref[...] = h * dinv


def _mid1_call(acc, g, dinv, b, W):
    return pl.pallas_call(
        _mid1_body,
        grid=(GRID,),
        in_specs=[
            pl.BlockSpec((NC, NB, FW), lambda i: (0, i, 0)),
            pl.BlockSpec((NB, FW), lambda i: (i, 0)),
            pl.BlockSpec((NB, 1), lambda i: (i, 0)),
            pl.BlockSpec((1, 64), lambda i: (0, 0)),
            pl.BlockSpec((64, 128), lambda i: (0, 0)),
        ],
        out_specs=pl.BlockSpec((NB, FW), lambda i: (i, 0)),
        out_shape=jax.ShapeDtypeStruct((NN, FW), jnp.float32),
    )(acc, g, dinv, b, W)


def _mid2_body(acc_ref, g_ref, dinv_ref, b_ref, w_ref, outa_ref, outb_ref):
    dinv = dinv_ref[...]
    a = acc_ref[0] + acc_ref[1] + g_ref[...]
    a = jnp.maximum(dinv * a + b_ref[...], 0.0)
    h = jnp.dot(a, w_ref[...], preferred_element_type=jnp.float32)
    gn = h * dinv
    outa_ref[...] = gn[:, :FW]
    outb_ref[...] = gn[:, FW:]


def _mid2_call(acc, g, dinv, b, W):
    return pl.pallas_call(
        _mid2_body,
        grid=(GRID,),
        in_specs=[
            pl.BlockSpec((NC, NB, FW), lambda i: (0, i, 0)),
            pl.BlockSpec((NB, FW), lambda i: (i, 0)),
            pl.BlockSpec((NB, 1), lambda i: (i, 0)),
            pl.BlockSpec((1, 128), lambda i: (0, 0)),
            pl.BlockSpec((128, 256), lambda i: (0, 0)),
        ],
        out_specs=[
            pl.BlockSpec((NB, FW), lambda i: (i, 0)),
            pl.BlockSpec((NB, FW), lambda i: (i, 0)),
        ],
        out_shape=[
            jax.ShapeDtypeStruct((NN, FW), jnp.float32),
            jax.ShapeDtypeStruct((NN, FW), jnp.float32),
        ],
    )(acc, g, dinv, b, W)


def _post_body(acc_ref, ga_ref, gb_ref, dinv_ref, b_ref, bat_ref, lw1_ref,
               lb1_ref, gam_ref, bet_ref, lw2_ref, lb2_ref, out_ref, pool_ref):
    i = pl.program_id(0)

    @pl.when(i == 0)
    def _init():
        pool_ref[...] = jnp.full((NSEG, 256), -jnp.inf, jnp.float32)

    dinv = dinv_ref[...]
    a = jnp.concatenate(
        [acc_ref[0] + ga_ref[...], acc_ref[1] + gb_ref[...]], axis=1)
    a = jnp.maximum(dinv * a + b_ref[...], 0.0)
    bb = bat_ref[...]
    lo = jnp.min(bb)
    hi = jnp.max(bb)

    def seg_body(gid, carry):
        m = jnp.max(jnp.where(bb == gid, a, -jnp.inf), axis=0)
        cur = pool_ref[pl.ds(gid, 1), :]
        pool_ref[pl.ds(gid, 1), :] = jnp.maximum(cur, m[None, :])
        return carry

    lax.fori_loop(lo, hi + 1, seg_body, 0)

    @pl.when(i == pl.num_programs(0) - 1)
    def _fin():
        pooled = pool_ref[...]
        pooled = jnp.where(pooled > -3e38, pooled, 0.0)
        z = jnp.dot(pooled, lw1_ref[...], preferred_element_type=jnp.float32)
        z = jnp.maximum(z + lb1_ref[...], 0.0)
        z = gam_ref[...] * (z * np.float32(1.0 / np.sqrt(1.0 + 1e-5))) + bet_ref[...]
        out_ref[...] = jnp.dot(z, lw2_ref[...], preferred_element_type=jnp.float32) + lb2_ref[...]


def _post_call(acc, ga, gb, dinv, b3, batch2, lw1, lb1, gamma, beta, lw2, lb2):
    return pl.pallas_call(
        _post_body,
        grid=(GRID,),
        in_specs=[
            pl.BlockSpec((NC, NB, FW), lambda i: (0, i, 0)),
            pl.BlockSpec((NB, FW), lambda i: (i, 0)),
            pl.BlockSpec((NB, FW), lambda i: (i, 0)),
            pl.BlockSpec((NB, 1), lambda i: (i, 0)),
            pl.BlockSpec((1, 256), lambda i: (0, 0)),
            pl.BlockSpec((NB, 1), lambda i: (i, 0)),
            pl.BlockSpec((256, 128), lambda i: (0, 0)),
            pl.BlockSpec((1, 128), lambda i: (0, 0)),
            pl.BlockSpec((1, 128), lambda i: (0, 0)),
            pl.BlockSpec((1, 128), lambda i: (0, 0)),
            pl.BlockSpec((128, 1), lambda i: (0, 0)),
            pl.BlockSpec((1, 1), lambda i: (0, 0)),
        ],
        out_specs=pl.BlockSpec((NSEG, 1), lambda i: (0, 0)),
        out_shape=jax.ShapeDtypeStruct((NSEG, 1), jnp.float32),
        scratch_shapes=[pltpu.VMEM((NSEG, 256), jnp.float32)],
        compiler_params=pltpu.CompilerParams(
            dimension_semantics=("arbitrary",)),
    )(acc, ga, gb, dinv, b3, batch2, lw1, lb1, gamma, beta, lw2, lb2)


def kernel(x, edge_index, batch, W1, b1, W2, b2, W3, b3,
           lw1, lb1, gamma, beta, lw2, lb2):
    src = edge_index[0].astype(jnp.int32)
    dst = edge_index[1].astype(jnp.int32)
    npad = EPAD - EE
    s_pad = jnp.concatenate([src, jnp.zeros((npad,), jnp.int32)])
    d_pad = jnp.concatenate([dst, jnp.full((npad,), NN, jnp.int32)])
    srcw_e = s_pad.reshape(NW, NSUP_E, SUP, CHUNK)
    dstw_e = d_pad.reshape(NW, NSUP_E, SUP, CHUNK)
    srcw_f = s_pad.reshape(NS, NSUP_F, SUP, CHUNK)
    dstw_f = d_pad.reshape(NS, NSUP_F, SUP, CHUNK)
    zer = jnp.zeros((CHUNK, FW), jnp.float32)
    ones = jnp.ones((CHUNK, FW), jnp.float32)

    degraw = _DEG(d_pad.reshape(NW, NCH_E, CHUNK), ones, zer)
    g1, dinv = _pre_call(x, W1, degraw)

    acc1 = _SCAT_E(g1, g1, srcw_e, dstw_e, zer)
    g2 = _mid1_call(acc1, g1, dinv, b1.reshape(1, 64), W2)

    acc2 = _SCAT_E(g2, g2, srcw_e, dstw_e, zer)
    g3a, g3b = _mid2_call(acc2, g2, dinv, b2.reshape(1, 128), W3)

    acc3 = _SCAT_F(g3a, g3b, srcw_f, dstw_f, zer)

    out = _post_call(acc3, g3a, g3b, dinv, b3.reshape(1, 256),
                     batch.reshape(NN, 1).astype(jnp.int32),
                     lw1, lb1.reshape(1, 128),
                     gamma.reshape(1, 128), beta.reshape(1, 128),
                     lw2, lb2.reshape(1, 1))
    return out.reshape(-1)
